# Initial kernel scaffold; baseline (speedup 1.0000x reference)
import jax
import jax.numpy as jnp
from jax.experimental import pallas as pl


def kernel(x, memory, W1, b1, W2, b2, W3, b3):
    raise NotImplementedError("write your pallas kernel here")


# speedup vs baseline: 2.2801x; 2.2801x over previous
import functools

import jax
import jax.numpy as jnp
from jax import lax
from jax.experimental import pallas as pl
from jax.experimental.pallas import tpu as pltpu
from jax.experimental.pallas import tpu_sc as plsc

BATCH = 4
SEQ = 8192
INPUT_DIM = 1024
SPARSITY = 0.5
CLUSTER = 4
HIDDEN = 256
MEM = 128
CLASSES = 10
K_TOK = int(SPARSITY * SEQ)
CLUST_IN = CLUSTER * INPUT_DIM
NUM_CLUSTERS = K_TOK // CLUSTER

_SROWS = SEQ // 128
_LOGN = 13


def _sort_body(key_ref, idx_out_ref):
    keys = key_ref[...]
    shape = keys.shape
    lane = lax.broadcasted_iota(jnp.int32, shape, 2)
    row = lax.broadcasted_iota(jnp.int32, shape, 1)
    pos = row * 128 + lane
    idx = pos

    def partner(arr, d, axis, ax_iota):
        take_minus = (ax_iota & d) != 0
        return jnp.where(take_minus, jnp.roll(arr, d, axis=axis),
                         jnp.roll(arr, -d, axis=axis))

    for k in range(1, _LOGN + 1):
        for j in reversed(range(k)):
            d = 1 << j
            if d < 128:
                pk = partner(keys, d, 2, lane)
                pi = partner(idx, d, 2, lane)
            else:
                m = d // 128
                pk = partner(keys, m, 1, row)
                pi = partner(idx, m, 1, row)
            prec = (keys > pk) | ((keys == pk) & (idx < pi))
            low = (pos & d) == 0
            up = (pos & (1 << k)) == 0
            keep = (low == prec) == up
            keys = jnp.where(keep, keys, pk)
            idx = jnp.where(keep, idx, pi)
    idx_out_ref[...] = idx


def _topk_idx(norms):
    idx = pl.pallas_call(
        _sort_body,
        in_specs=[pl.BlockSpec((BATCH, _SROWS, 128), lambda: (0, 0, 0))],
        out_specs=pl.BlockSpec((BATCH, _SROWS, 128), lambda: (0, 0, 0)),
        out_shape=jax.ShapeDtypeStruct((BATCH, _SROWS, 128), jnp.int32),
    )(norms.reshape(BATCH, _SROWS, 128))
    return idx.reshape(BATCH, SEQ)[:, :K_TOK]


_GCHUNK = 64


@functools.cache
def _make_sc_gather():
    nc, ns = 2, 16
    nw = nc * ns
    rows_total = BATCH * K_TOK
    rows_per_w = rows_total // nw
    nchunks = rows_per_w // _GCHUNK
    mesh = plsc.VectorSubcoreMesh(core_axis_name="c", subcore_axis_name="s",
                                  num_cores=nc, num_subcores=ns)

    @functools.partial(
        pl.kernel,
        mesh=mesh,
        out_type=jax.ShapeDtypeStruct((rows_total, INPUT_DIM), jnp.float32),
        scratch_types=[
            pltpu.VMEM((_GCHUNK,), jnp.int32),
            pltpu.VMEM((_GCHUNK, INPUT_DIM), jnp.float32),
            pltpu.SemaphoreType.DMA,
        ],
    )
    def sc_gather(table_hbm, idx_hbm, out_hbm, idx_v, rows_v, sem):
        wid = lax.axis_index("s") * nc + lax.axis_index("c")
        base = wid * rows_per_w
        for c in range(nchunks):
            off = base + c * _GCHUNK
            pltpu.sync_copy(idx_hbm.at[pl.ds(off, _GCHUNK)], idx_v)
            pltpu.async_copy(table_hbm.at[idx_v], rows_v, sem).wait()
            pltpu.sync_copy(rows_v, out_hbm.at[pl.ds(off, _GCHUNK)])

    return sc_gather

_CB = 128
_NCB = NUM_CLUSTERS // _CB


def _dense_body(cl_ref, w1_ref, b1_ref, mask_ref, mem_ref, w2_ref, b2_ref,
                w3_ref, b3_ref, out_ref, um_ref, acc_ref):
    k = pl.program_id(0)
    cb = cl_ref[...]
    h = lax.dot_general(cb, w1_ref[...], (((2,), (0,)), ((), ())),
                        preferred_element_type=jnp.float32)
    h = h + b1_ref[...][None]
    h = jnp.maximum(h * mask_ref[...], 0.0)
    psum = jnp.sum(h, axis=1)

    @pl.when(k == 0)
    def _():
        acc_ref[...] = psum

    @pl.when(k > 0)
    def _():
        acc_ref[...] = acc_ref[...] + psum

    @pl.when(k == _NCB - 1)
    def _():
        xc = acc_ref[...] * (1.0 / NUM_CLUSTERS)
        um = jnp.maximum(
            lax.dot_general(xc, w2_ref[...], (((1,), (0,)), ((), ())),
                            preferred_element_type=jnp.float32)
            + b2_ref[...] + mem_ref[...], 0.0)
        um_ref[...] = um
        out_ref[...] = (
            lax.dot_general(um, w3_ref[...], (((1,), (0,)), ((), ())),
                            preferred_element_type=jnp.float32)
            + b3_ref[...])


def _dense(clustered, W1, b1, mask, memory, W2, b2, W3, b3):
    return pl.pallas_call(
        _dense_body,
        grid=(_NCB,),
        in_specs=[
            pl.BlockSpec((BATCH, _CB, CLUST_IN), lambda k: (0, k, 0)),
            pl.BlockSpec((CLUST_IN, HIDDEN), lambda k: (0, 0)),
            pl.BlockSpec((1, HIDDEN), lambda k: (0, 0)),
            pl.BlockSpec((BATCH, _CB, HIDDEN), lambda k: (0, k, 0)),
            pl.BlockSpec((BATCH, MEM), lambda k: (0, 0)),
            pl.BlockSpec((HIDDEN, MEM), lambda k: (0, 0)),
            pl.BlockSpec((1, MEM), lambda k: (0, 0)),
            pl.BlockSpec((MEM, CLASSES), lambda k: (0, 0)),
            pl.BlockSpec((1, CLASSES), lambda k: (0, 0)),
        ],
        out_specs=[
            pl.BlockSpec((BATCH, CLASSES), lambda k: (0, 0)),
            pl.BlockSpec((BATCH, MEM), lambda k: (0, 0)),
        ],
        out_shape=[
            jax.ShapeDtypeStruct((BATCH, CLASSES), jnp.float32),
            jax.ShapeDtypeStruct((BATCH, MEM), jnp.float32),
        ],
        scratch_shapes=[pltpu.VMEM((BATCH, HIDDEN), jnp.float32)],
        compiler_params=pltpu.CompilerParams(
            dimension_semantics=("arbitrary",)),
    )(clustered, W1, b1, mask, memory, W2, b2, W3, b3)


def kernel(x, memory, W1, b1, W2, b2, W3, b3):
    norms = jnp.sqrt(jnp.sum(x * x, axis=-1))
    topk_idx = _topk_idx(norms)
    gidx = (topk_idx.astype(jnp.int32)
            + (jnp.arange(BATCH, dtype=jnp.int32) * SEQ)[:, None]).reshape(-1)
    staged = _make_sc_gather()(x.reshape(BATCH * SEQ, INPUT_DIM), gidx)
    clustered = staged.reshape(BATCH, NUM_CLUSTERS, CLUST_IN)
    mask = jax.random.bernoulli(
        jax.random.key(1), SPARSITY,
        (BATCH, NUM_CLUSTERS, HIDDEN)).astype(jnp.float32)
    out, um = _dense(clustered, W1, b1.reshape(1, HIDDEN), mask, memory,
                     W2, b2.reshape(1, MEM), W3, b3.reshape(1, CLASSES))
    return (out, um)


# speedup vs baseline: 2.3301x; 1.0219x over previous
import functools

import jax
import jax.numpy as jnp
import numpy as np
from jax import lax
from jax.experimental import pallas as pl
from jax.experimental.pallas import tpu as pltpu
from jax.experimental.pallas import tpu_sc as plsc

BATCH = 4
SEQ = 8192
INPUT_DIM = 1024
SPARSITY = 0.5
CLUSTER = 4
HIDDEN = 256
MEM = 128
CLASSES = 10
K_TOK = int(SPARSITY * SEQ)
CLUST_IN = CLUSTER * INPUT_DIM
NUM_CLUSTERS = K_TOK // CLUSTER

_SROWS = SEQ // 128
_LOGN = 13


def _sort_body(key_ref, idx_out_ref):
    keys = key_ref[...]
    shape = keys.shape
    lane = lax.broadcasted_iota(jnp.int32, shape, 2)
    row = lax.broadcasted_iota(jnp.int32, shape, 1)
    pos = row * 128 + lane
    idx = pos

    def partner(arr, d, axis, ax_iota):
        take_minus = (ax_iota & d) != 0
        return jnp.where(take_minus, jnp.roll(arr, d, axis=axis),
                         jnp.roll(arr, -d, axis=axis))

    for k in range(1, _LOGN + 1):
        for j in reversed(range(k)):
            d = 1 << j
            if d < 128:
                pk = partner(keys, d, 2, lane)
                pi = partner(idx, d, 2, lane)
            else:
                m = d // 128
                pk = partner(keys, m, 1, row)
                pi = partner(idx, m, 1, row)
            prec = (keys > pk) | ((keys == pk) & (idx < pi))
            low = (pos & d) == 0
            up = (pos & (1 << k)) == 0
            keep = (low == prec) == up
            keys = jnp.where(keep, keys, pk)
            idx = jnp.where(keep, idx, pi)
    bofs = lax.broadcasted_iota(jnp.int32, (BATCH, _SROWS // 2, 128), 0) * SEQ
    idx_out_ref[...] = idx[:, : _SROWS // 2, :] + bofs


def _topk_gidx(norms):
    idx = pl.pallas_call(
        _sort_body,
        in_specs=[pl.BlockSpec((BATCH, _SROWS, 128), lambda: (0, 0, 0))],
        out_specs=pl.BlockSpec((BATCH, _SROWS // 2, 128), lambda: (0, 0, 0)),
        out_shape=jax.ShapeDtypeStruct((BATCH, _SROWS // 2, 128), jnp.int32),
    )(norms.reshape(BATCH, _SROWS, 128))
    return idx.reshape(BATCH * K_TOK)


_GCHUNK = 32


@functools.cache
def _make_sc_gather():
    nc, ns = 2, 16
    nw = nc * ns
    rows_total = BATCH * K_TOK
    rows_per_w = rows_total // nw
    nchunks = rows_per_w // _GCHUNK
    mesh = plsc.VectorSubcoreMesh(core_axis_name="c", subcore_axis_name="s",
                                  num_cores=nc, num_subcores=ns)

    @functools.partial(
        pl.kernel,
        mesh=mesh,
        out_type=jax.ShapeDtypeStruct((rows_total, INPUT_DIM), jnp.float32),
        scratch_types=[
            pltpu.VMEM((nchunks, _GCHUNK), jnp.int32),
            pltpu.VMEM((_GCHUNK, INPUT_DIM), jnp.float32),
            pltpu.VMEM((_GCHUNK, INPUT_DIM), jnp.float32),
            pltpu.SemaphoreType.DMA,
            pltpu.SemaphoreType.DMA,
            pltpu.SemaphoreType.DMA,
            pltpu.SemaphoreType.DMA,
        ],
    )
    def sc_gather(table_hbm, idx_hbm, out_hbm, idx_v, rows0, rows1,
                  g0, g1, w0, w1):
        wid = lax.axis_index("s") * nc + lax.axis_index("c")
        base = wid * rows_per_w
        rows = (rows0, rows1)
        gsem = (g0, g1)
        wsem = (w0, w1)
        pltpu.sync_copy(idx_hbm.at[wid], idx_v)
        gathers = [None, None]
        writes = [None, None]
        for c in range(nchunks):
            cur = c % 2
            if c == 0:
                gathers[0] = pltpu.async_copy(
                    table_hbm.at[idx_v.at[0]], rows0, g0)
            gathers[cur].wait()
            if c + 1 < nchunks:
                nxt = (c + 1) % 2
                if writes[nxt] is not None:
                    writes[nxt].wait()
                gathers[nxt] = pltpu.async_copy(
                    table_hbm.at[idx_v.at[c + 1]], rows[nxt], gsem[nxt])
            writes[cur] = pltpu.async_copy(
                rows[cur], out_hbm.at[pl.ds(base + c * _GCHUNK, _GCHUNK)],
                wsem[cur])
        writes[0].wait()
        writes[1].wait()

    return sc_gather

_CB = 128
_NCB = NUM_CLUSTERS // _CB


def _dense_body(cl_ref, w1_ref, b1_ref, mask_ref, mem_ref, w2_ref, b2_ref,
                w3_ref, b3_ref, out_ref, um_ref, acc_ref):
    k = pl.program_id(0)
    cb = cl_ref[...]
    h = lax.dot_general(cb, w1_ref[...], (((2,), (0,)), ((), ())),
                        preferred_element_type=jnp.float32)
    h = h + b1_ref[...][None]
    h = jnp.maximum(h * mask_ref[...], 0.0)
    psum = jnp.sum(h, axis=1)

    @pl.when(k == 0)
    def _():
        acc_ref[...] = psum

    @pl.when(k > 0)
    def _():
        acc_ref[...] = acc_ref[...] + psum

    @pl.when(k == _NCB - 1)
    def _():
        xc = acc_ref[...] * (1.0 / NUM_CLUSTERS)
        um = jnp.maximum(
            lax.dot_general(xc, w2_ref[...], (((1,), (0,)), ((), ())),
                            preferred_element_type=jnp.float32)
            + b2_ref[...] + mem_ref[...], 0.0)
        um_ref[...] = um
        out_ref[...] = (
            lax.dot_general(um, w3_ref[...], (((1,), (0,)), ((), ())),
                            preferred_element_type=jnp.float32)
            + b3_ref[...])


def _dense(clustered, W1, b1, mask, memory, W2, b2, W3, b3):
    return pl.pallas_call(
        _dense_body,
        grid=(_NCB,),
        in_specs=[
            pl.BlockSpec((BATCH, _CB, CLUST_IN), lambda k: (0, k, 0)),
            pl.BlockSpec((CLUST_IN, HIDDEN), lambda k: (0, 0)),
            pl.BlockSpec((1, HIDDEN), lambda k: (0, 0)),
            pl.BlockSpec((BATCH, _CB, HIDDEN), lambda k: (0, k, 0)),
            pl.BlockSpec((BATCH, MEM), lambda k: (0, 0)),
            pl.BlockSpec((HIDDEN, MEM), lambda k: (0, 0)),
            pl.BlockSpec((1, MEM), lambda k: (0, 0)),
            pl.BlockSpec((MEM, CLASSES), lambda k: (0, 0)),
            pl.BlockSpec((1, CLASSES), lambda k: (0, 0)),
        ],
        out_specs=[
            pl.BlockSpec((BATCH, CLASSES), lambda k: (0, 0)),
            pl.BlockSpec((BATCH, MEM), lambda k: (0, 0)),
        ],
        out_shape=[
            jax.ShapeDtypeStruct((BATCH, CLASSES), jnp.float32),
            jax.ShapeDtypeStruct((BATCH, MEM), jnp.float32),
        ],
        scratch_shapes=[pltpu.VMEM((BATCH, HIDDEN), jnp.float32)],
        compiler_params=pltpu.CompilerParams(
            dimension_semantics=("arbitrary",)),
    )(clustered, W1, b1, mask, memory, W2, b2, W3, b3)


@functools.cache
def _mask_const():
    with jax.ensure_compile_time_eval():
        m = jax.random.bernoulli(
            jax.random.key(1), SPARSITY,
            (BATCH, NUM_CLUSTERS, HIDDEN)).astype(jnp.float32)
    return np.asarray(m)


def kernel(x, memory, W1, b1, W2, b2, W3, b3):
    norms = jnp.sqrt(jnp.sum(x * x, axis=-1))
    gidx = _topk_gidx(norms)
    nw, nch = 32, 16
    staged = _make_sc_gather()(
        x.reshape(BATCH * SEQ, INPUT_DIM),
        gidx.reshape(nw, nch, _GCHUNK))
    clustered = staged.reshape(BATCH, NUM_CLUSTERS, CLUST_IN)
    mask = jnp.asarray(_mask_const())
    out, um = _dense(clustered, W1, b1.reshape(1, HIDDEN), mask, memory,
                     W2, b2.reshape(1, MEM), W3, b3.reshape(1, CLASSES))
    return (out, um)
